# baseline (speedup 1.0000x reference)
import functools

import jax
import jax.numpy as jnp
from jax import lax
from jax.experimental import pallas as pl
from jax.experimental.pallas import tpu as pltpu
from jax.experimental.pallas import tpu_sc as plsc

L = 16
B = 2048
CS = 8192
NBUF = 4


def _sc_hist_kernel(nc, ns, w, rows_per_term, nch):
    mesh = plsc.VectorSubcoreMesh(core_axis_name="c", subcore_axis_name="s")
    rows = CS // w
    img_per_sc = 4 // nc
    trows_sc = img_per_sc * rows_per_term

    @functools.partial(
        pl.kernel,
        out_type=jax.ShapeDtypeStruct((32, L), jnp.float32),
        mesh=mesh,
        compiler_params=pltpu.CompilerParams(needs_layout_passes=False),
        scratch_types=[
            *[pltpu.VMEM((rows, w), jnp.float32) for _ in range(NBUF)],
            *[pltpu.VMEM((rows, w), jnp.int32) for _ in range(NBUF)],
            pltpu.VMEM((2 * B,), jnp.float32),
            pltpu.VMEM((L,), jnp.float32),
            pltpu.VMEM_SHARED((trows_sc, w), jnp.int32),
            *[pltpu.SemaphoreType.DMA for _ in range(2 * NBUF)],
        ],
    )
    def hist_kernel(p_hbm, t_hbm, out_hbm, *rest):
        pbufs = rest[0:NBUF]
        tbufs = rest[NBUF:2 * NBUF]
        hist = rest[2 * NBUF]
        res = rest[2 * NBUF + 1]
        shared_t = rest[2 * NBUF + 2]
        psems = rest[2 * NBUF + 3:2 * NBUF + 3 + NBUF]
        tsems = rest[2 * NBUF + 3 + NBUF:2 * NBUF + 3 + 2 * NBUF]
        cid = lax.axis_index("c")
        sid = lax.axis_index("s")
        wid = cid * ns + sid
        img = wid // 8
        cls = wid % 8

        zeros = jnp.zeros((L,), jnp.float32)
        ones = jnp.ones((L,), jnp.float32)
        fB = jnp.float32(B)

        prow = wid * rows_per_term
        trow_local = (img - img_per_sc * cid) * rows_per_term

        def issue_p(ch, b):
            pltpu.async_copy(
                p_hbm.at[pl.ds(prow + ch * rows, rows), :], pbufs[b], psems[b])

        def issue_t(ch, b):
            pltpu.async_copy(
                shared_t.at[pl.ds(trow_local + ch * rows, rows), :],
                tbufs[b], tsems[b])

        def issue(ch, b):
            issue_p(ch, b)
            issue_t(ch, b)

        def wait(b):
            pltpu.make_async_copy(
                p_hbm.at[pl.ds(0, rows), :], pbufs[b], psems[b]).wait()
            pltpu.make_async_copy(
                shared_t.at[pl.ds(0, rows), :], tbufs[b], tsems[b]).wait()

        for b in range(NBUF):
            issue_p(b, b)
        srows = trows_sc // ns
        pltpu.sync_copy(
            t_hbm.at[pl.ds(cid * trows_sc + sid * srows, srows), :],
            shared_t.at[pl.ds(sid * srows, srows), :])

        def zbody(j, _):
            hist[pl.ds(j * L, L)] = zeros
            return 0
        lax.fori_loop(0, (2 * B) // L, zbody, 0)

        plsc.subcore_barrier()
        for b in range(NBUF):
            issue_t(b, b)

        def chunk_group(g, _):
            for b in range(NBUF):
                ch = g * NBUF + b
                wait(b)
                pb = pbufs[b]
                tb = tbufs[b]

                wshift = w.bit_length() - 1

                @plsc.parallel_loop(0, CS // L, step=1, unroll=16)
                def _3(k):
                    i = k * L
                    r = lax.shift_right_logical(i, wshift)
                    col = lax.bitwise_and(i, w - 1)
                    p = pb[r, pl.ds(col, L)]
                    t = tb[r, pl.ds(col, L)]
                    m = t == cls
                    v = jnp.where(m, 2.0 - p, p)
                    iy = (v * fB).astype(jnp.int32)
                    iy = jnp.minimum(iy, 2 * B - 1)
                    plsc.addupdate_scatter(hist, [iy], ones)

                nxt = ch + NBUF

                @pl.when(nxt < nch)
                def _():
                    issue(nxt, b)
            return 0

        lax.fori_loop(0, nch // NBUF, chunk_group, 0)

        nq = B // (4 * L)
        pq = 3 * B // (4 * L)

        def gbody(j, carry):
            gv, cv = carry
            hpc = hist[pl.ds(B + j * L, L)]
            hnc = hist[pl.ds(j * L, L)]
            mn = (j >= nq).astype(jnp.float32)
            mp = (j < pq).astype(jnp.float32)
            return (gv + hpc, cv + hnc * mn + hpc * mp)

        gv, cv = lax.fori_loop(0, B // L, gbody, (zeros, zeros))
        G = jnp.sum(gv)
        cnt25 = jnp.sum(cv)

        def sbody(j, carry):
            jacc, pc, kc = carry
            asc = B - L * (j + 1)
            hp = lax.rev(hist[pl.ds(B + asc, L)], (0,))
            hn = lax.rev(hist[pl.ds(asc, L)], (0,))
            n = hp + hn
            pv = pc + plsc.cumsum(hp)
            kv = kc + plsc.cumsum(n)
            jbin = jnp.where(
                kv > 0.0,
                1.0 - (G - pv) / jnp.maximum(G + kv - pv, 1.0),
                0.0,
            )
            return (jacc + jbin, pc + jnp.sum(hp), kc + jnp.sum(n))

        jacc, _, _ = lax.fori_loop(
            0, B // L, sbody, (zeros, jnp.float32(0.0), jnp.float32(0.0)))
        term = (jnp.sum(jacc) - 0.5) * jnp.float32(1.0 / B)

        lane = lax.iota(jnp.int32, L)
        res[...] = jnp.where(
            lane == 0, term,
            jnp.where(lane == 1, G, jnp.where(lane == 2, cnt25, 0.0)))
        pltpu.sync_copy(res, out_hbm.at[wid])

    return hist_kernel


def _finalize_body(res_ref, aux_ref, out_ref):
    res = res_ref[...]
    aux = aux_ref[...]
    loss_t = res[:, 0:1]
    g = res[:, 1:2]
    c25 = res[:, 2:3]
    active = (aux[:, 1:2] != 0.0) & ((g > 0.0) | (c25 > 0.0))
    flag = active.astype(jnp.float32)
    total = jnp.sum(loss_t * aux[:, 0:1] * flag)
    ne = jnp.sum(flag)
    out_ref[...] = (total / 4.0 / ne)[None, None]


def kernel(inputs, targets, classes_weights, tiles_weights, config):
    n, c_dim, h, w = inputs.shape
    hw = h * w
    nch = hw // CS

    info = plsc.get_sparse_core_info()
    nc = info.num_cores

    p2 = inputs.reshape(n * c_dim * h, w)
    t2 = targets.reshape(n * h, w)

    res32 = _sc_hist_kernel(nc, info.num_subcores, w, h, nch)(p2, t2)

    cw_full = jnp.tile(classes_weights, n)
    tw_full = jnp.repeat(tiles_weights, c_dim)
    aux = jnp.zeros((32, L), jnp.float32)
    aux = aux.at[:, 0].set(cw_full * tw_full)
    aux = aux.at[:, 1].set(cw_full)

    out = pl.pallas_call(
        _finalize_body,
        out_shape=jax.ShapeDtypeStruct((1, 1), jnp.float32),
    )(res32, aux)
    return out[0, 0]
